# baseline (speedup 1.0000x reference)
import functools

import jax
import jax.numpy as jnp
from jax import lax
from jax.experimental import pallas as pl
from jax.experimental.pallas import tpu as pltpu
from jax.experimental.pallas import tpu_sc as plsc

NU_REAL = 50001
NI_REAL = 50000
H = 64
HH = 32
E = 800000

NC = 2
NS = 16

NPAD = 50176
STRIPE = NPAD // NS
ZROWS = 784
E_TILE = 50176
EPAD = E_TILE * NS
CHUNK = 64
NCHUNK = E_TILE // CHUNK


NBUF = 8
SBCH = NBUF * CHUNK
NSB = E_TILE // SBCH
ER = EPAD // CHUNK
TROWS = E_TILE // CHUNK


def _spmm_once(c, s, x, rr, cc, vv, yy, zz, acc, ibr, ibc, ibv, gath,
               sem_i, sem_g, sem_s):
  pltpu.sync_copy(zz, acc.at[pl.ds(s * STRIPE, STRIPE)])
  plsc.subcore_barrier()

  def issue_idx(sb, bb):
    r0 = s * TROWS + sb * NBUF
    pltpu.async_copy(rr.at[pl.ds(r0, NBUF)], ibr.at[bb], sem_i)
    pltpu.async_copy(vv.at[pl.ds(r0, NBUF)], ibv.at[bb], sem_i)
    pltpu.async_copy(cc.at[pl.ds(r0, NBUF)], ibc.at[bb], sem_i)

  def wait_idx(bb):
    pltpu.make_async_copy(rr.at[pl.ds(0, NBUF)], ibr.at[bb], sem_i).wait()
    pltpu.make_async_copy(vv.at[pl.ds(0, NBUF)], ibv.at[bb], sem_i).wait()
    pltpu.make_async_copy(cc.at[pl.ds(0, NBUF)], ibc.at[bb], sem_i).wait()

  def wait_scatter(j):
    pltpu.make_async_copy(gath.at[j], acc.at[ibr.at[0, j]],
                          sem_s.at[j]).wait()

  issue_idx(0, 0)

  def sb_body(sb, carry):
    bb = lax.rem(sb, 2)
    wait_idx(bb)

    @pl.when(sb < NSB - 1)
    def _():
      issue_idx(sb + 1, 1 - bb)

    for j in range(NBUF):
      @pl.when(sb > 0)
      def _():
        wait_scatter(j)

      @pl.when(c == 0)
      def _():
        pltpu.async_copy(x.at[0].at[ibc.at[bb, j]], gath.at[j], sem_g.at[j])

      @pl.when(c == 1)
      def _():
        pltpu.async_copy(x.at[1].at[ibc.at[bb, j]], gath.at[j], sem_g.at[j])

    for j in range(NBUF):
      pltpu.make_async_copy(x.at[0].at[ibc.at[bb, j]], gath.at[j],
                            sem_g.at[j]).wait()

      def scale16(t, carry2):
        vv16 = ibv[bb, j, pl.ds(t * 16, 16)]
        for u in range(16):
          e = t * 16 + u
          val = vv16[u]
          gath[j, e, pl.ds(0, 16)] = gath[j, e, pl.ds(0, 16)] * val
          gath[j, e, pl.ds(16, 16)] = gath[j, e, pl.ds(16, 16)] * val
        return carry2

      lax.fori_loop(0, CHUNK // 16, scale16, 0)
      pltpu.async_copy(gath.at[j], acc.at[ibr.at[bb, j]], sem_s.at[j],
                       add=True)
    return carry

  lax.fori_loop(0, NSB, sb_body, 0)
  for j in range(NBUF):
    wait_scatter(j)
  plsc.subcore_barrier()
  @pl.when(c == 0)
  def _():
    pltpu.sync_copy(acc.at[pl.ds(s * STRIPE, STRIPE)],
                    yy.at[0, pl.ds(s * STRIPE, STRIPE)])

  @pl.when(c == 1)
  def _():
    pltpu.sync_copy(acc.at[pl.ds(s * STRIPE, STRIPE)],
                    yy.at[1, pl.ds(s * STRIPE, STRIPE)])

  plsc.subcore_barrier()


def _sc_pair_body(xu, xi, zz, r1, c1, v1, r2, c2, v2,
                  y1, y2, acc, ibr, ibc, ibv, gath,
                  sem_i, sem_g, sem_s):
  c = lax.axis_index("c")
  s = lax.axis_index("s")
  args = (acc, ibr, ibc, ibv, gath, sem_i, sem_g, sem_s)
  _spmm_once(c, s, xu, r1, c1, v1, y1, zz, *args)
  _spmm_once(c, s, xi, r2, c2, v2, y2, zz, *args)


def _sc_one_body(xu, zz, r3, c3, v3, y3, acc, ibr, ibc, ibv, gath,
                 sem_i, sem_g, sem_s):
  c = lax.axis_index("c")
  s = lax.axis_index("s")
  args = (acc, ibr, ibc, ibv, gath, sem_i, sem_g, sem_s)
  _spmm_once(c, s, xu, r3, c3, v3, y3, zz, *args)


_Y = jax.ShapeDtypeStruct((NC, NPAD, HH), jnp.float32)
_MESH = plsc.VectorSubcoreMesh(core_axis_name="c", subcore_axis_name="s",
                               num_cores=NC, num_subcores=NS)
_SCRATCH = [
    pltpu.VMEM_SHARED((NPAD, HH), jnp.float32),
    pltpu.VMEM((2, NBUF, CHUNK), jnp.int32),
    pltpu.VMEM((2, NBUF, CHUNK), jnp.int32),
    pltpu.VMEM((2, NBUF, CHUNK), jnp.float32),
    pltpu.VMEM((NBUF, CHUNK, HH), jnp.float32),
    pltpu.SemaphoreType.DMA,
    pltpu.SemaphoreType.DMA((NBUF,)),
    pltpu.SemaphoreType.DMA((NBUF,)),
]

_sc_pair = pl.kernel(
    _sc_pair_body,
    out_type=(_Y, _Y),
    mesh=_MESH,
    scratch_types=list(_SCRATCH),
    compiler_params=pltpu.CompilerParams(use_tc_tiling_on_sc=False),
)

_sc_one = pl.kernel(
    _sc_one_body,
    out_type=_Y,
    mesh=_MESH,
    scratch_types=list(_SCRATCH),
    compiler_params=pltpu.CompilerParams(use_tc_tiling_on_sc=False),
)


_BR = 1568
_R = NPAD // _BR


def _sums_body(x_ref, o_ref, acc_ref):
  r = pl.program_id(0)

  @pl.when(r == 0)
  def _():
    acc_ref[0] = 0.0
    acc_ref[1] = 0.0

  x = x_ref[...]
  acc_ref[0] += jnp.sum(x)
  acc_ref[1] += jnp.sum(x * x)

  @pl.when(r == _R - 1)
  def _():
    o_ref[0] = acc_ref[0]
    o_ref[1] = acc_ref[1]


def _apply_norm_body(ms_ref, x_ref, o_ref):
  mean = ms_ref[0, 0]
  scale = ms_ref[0, 1]
  y = (x_ref[...] - mean) * scale
  o_ref[0] = y[:, :HH]
  o_ref[1] = y[:, HH:]


def _normalize_split(x_pad, count):
  sums = pl.pallas_call(
      _sums_body,
      grid=(_R,),
      in_specs=[pl.BlockSpec((_BR, H), lambda r: (r, 0))],
      out_specs=pl.BlockSpec(memory_space=pltpu.SMEM),
      out_shape=jax.ShapeDtypeStruct((2,), jnp.float32),
      scratch_shapes=[pltpu.SMEM((2,), jnp.float32)],
  )(x_pad)
  mean = sums[0] / count
  var = sums[1] / count - mean * mean
  ms = jnp.stack([mean, 0.1 * lax.rsqrt(var)]).reshape(1, 2)
  return pl.pallas_call(
      _apply_norm_body,
      grid=(_R,),
      in_specs=[pl.BlockSpec(memory_space=pltpu.SMEM),
                pl.BlockSpec((_BR, H), lambda r: (r, 0))],
      out_specs=pl.BlockSpec((NC, _BR, HH), lambda r: (0, r, 0)),
      out_shape=jax.ShapeDtypeStruct((NC, NPAD, HH), jnp.float32),
  )(ms, x_pad)


def _attw(att, h0_ref, h1_ref, h2_ref):
  p0 = jnp.concatenate([h0_ref[0], h0_ref[1]], axis=1)
  p1 = jnp.concatenate([h1_ref[0], h1_ref[1]], axis=1)
  p2 = jnp.concatenate([h2_ref[0], h2_ref[1]], axis=1)
  cat = jnp.concatenate([p0, p1, p2], axis=1)
  logits = jnp.dot(cat, att, preferred_element_type=jnp.float32)
  m = jnp.max(logits, axis=1, keepdims=True)
  ex = jnp.exp(logits - m)
  w = ex / jnp.sum(ex, axis=1, keepdims=True)
  return w, p0, p1, p2


def _fuse_u_body(att_ref, u0_ref, us_ref, ui_ref, u1_ref):
  w, _, _, _ = _attw(att_ref[...], u0_ref, us_ref, ui_ref)
  for cc in range(NC):
    u1_ref[cc] = (w[:, 0:1] * u0_ref[cc] + w[:, 1:2] * us_ref[cc]
                  + w[:, 2:3] * ui_ref[cc])


def _fuse_u(att1, u0, us1, ui1):
  half = pl.BlockSpec((NC, _BR, HH), lambda r: (0, r, 0))
  return pl.pallas_call(
      _fuse_u_body,
      grid=(_R,),
      in_specs=[pl.BlockSpec((3 * H, 3), lambda r: (0, 0)),
                half, half, half],
      out_specs=half,
      out_shape=jax.ShapeDtypeStruct((NC, NPAD, HH), jnp.float32),
  )(att1, u0, us1, ui1)


def _fuse_i_body(i0_ref, iu_ref, i1_ref):
  for cc in range(NC):
    i1_ref[cc] = 0.5 * (i0_ref[cc] + iu_ref[cc])


def _fuse_i(i0, iu1):
  half = pl.BlockSpec((NC, _BR, HH), lambda r: (0, r, 0))
  return pl.pallas_call(
      _fuse_i_body,
      grid=(_R,),
      in_specs=[half, half],
      out_specs=half,
      out_shape=jax.ShapeDtypeStruct((NC, NPAD, HH), jnp.float32),
  )(i0, iu1)


def _final_u_body(att_ref, w_ref, b_ref, u1_ref, us_ref, ui_ref, fu_ref):
  wr = w_ref[...]
  b = b_ref[...]
  w, u1f, usf, uif = _attw(att_ref[...], u1_ref, us_ref, ui_ref)
  u2f = w[:, 0:1] * u1f + w[:, 1:2] * usf + w[:, 2:3] * uif
  zu = (jnp.dot(u1f, wr[:H], preferred_element_type=jnp.float32)
        + jnp.dot(u2f, wr[H:], preferred_element_type=jnp.float32) + b)
  fu_ref[...] = 1.0 / (1.0 + jnp.exp(-zu))


def _final_u(att2, w_reduce, b_reduce, u1, us2, ui2):
  half = pl.BlockSpec((NC, _BR, HH), lambda r: (0, r, 0))
  full = pl.BlockSpec((_BR, H), lambda r: (r, 0))
  return pl.pallas_call(
      _final_u_body,
      grid=(_R,),
      in_specs=[pl.BlockSpec((3 * H, 3), lambda r: (0, 0)),
                pl.BlockSpec((2 * H, H), lambda r: (0, 0)),
                pl.BlockSpec((1, H), lambda r: (0, 0)),
                half, half, half],
      out_specs=full,
      out_shape=jax.ShapeDtypeStruct((NPAD, H), jnp.float32),
  )(att2, w_reduce, b_reduce, u1, us2, ui2)


def _final_i_body(w_ref, b_ref, i1_ref, iu_ref, fi_ref):
  wr = w_ref[...]
  b = b_ref[...]
  i1f = jnp.concatenate([i1_ref[0], i1_ref[1]], axis=1)
  iuf = jnp.concatenate([iu_ref[0], iu_ref[1]], axis=1)
  i2f = 0.5 * (i1f + iuf)
  zi = (jnp.dot(i1f, wr[:H], preferred_element_type=jnp.float32)
        + jnp.dot(i2f, wr[H:], preferred_element_type=jnp.float32) + b)
  fi_ref[...] = 1.0 / (1.0 + jnp.exp(-zi))


def _final_i(w_reduce, b_reduce, i1, iu2):
  half = pl.BlockSpec((NC, _BR, HH), lambda r: (0, r, 0))
  full = pl.BlockSpec((_BR, H), lambda r: (r, 0))
  return pl.pallas_call(
      _final_i_body,
      grid=(_R,),
      in_specs=[pl.BlockSpec((2 * H, H), lambda r: (0, 0)),
                pl.BlockSpec((1, H), lambda r: (0, 0)),
                half, half],
      out_specs=full,
      out_shape=jax.ShapeDtypeStruct((NPAD, H), jnp.float32),
  )(w_reduce, b_reduce, i1, iu2)


def _prep_edges(idx, vals):
  rows = idx[:, 0].astype(jnp.int32)
  cols = idx[:, 1].astype(jnp.int32)
  pad = EPAD - E
  rows_p = jnp.concatenate([rows, jnp.full((pad,), NPAD - 1, jnp.int32)])
  cols_p = jnp.concatenate([cols, jnp.zeros((pad,), jnp.int32)])
  vals_p = jnp.concatenate([vals, jnp.zeros((pad,), jnp.float32)])
  return (rows_p.reshape(ER, CHUNK), cols_p.reshape(ER, CHUNK),
          vals_p.reshape(ER, CHUNK))


def kernel(user_embedding, item_embedding, att1, att2, W_reduce, b_reduce,
           social_neighbors_indices, social_neighbors_values,
           consumed_items_indices, consumed_items_values,
           item_customer_indices, item_customer_values):
  ue = jnp.zeros((NPAD, H), jnp.float32).at[:NU_REAL].set(user_embedding)
  ie = jnp.zeros((NPAD, H), jnp.float32).at[:NI_REAL].set(item_embedding)

  u0 = _normalize_split(ue, float(NU_REAL * H))
  i0 = _normalize_split(ie, float(NI_REAL * H))

  r1, c1, v1 = _prep_edges(social_neighbors_indices, social_neighbors_values)
  r2, c2, v2 = _prep_edges(consumed_items_indices, consumed_items_values)
  r3, c3, v3 = _prep_edges(item_customer_indices, item_customer_values)

  zz = jnp.zeros((STRIPE, HH), jnp.float32)
  iu1 = _sc_one(u0, zz, r3, c3, v3)
  us1, ui1 = _sc_pair(u0, i0, zz, r1, c1, v1, r2, c2, v2)
  i1 = _fuse_i(i0, iu1)
  u1 = _fuse_u(att1, u0, us1, ui1)

  iu2 = _sc_one(u1, zz, r3, c3, v3)
  us2, ui2 = _sc_pair(u1, i1, zz, r1, c1, v1, r2, c2, v2)
  fi = _final_i(W_reduce, b_reduce.reshape(1, H), i1, iu2)
  fu = _final_u(att2, W_reduce, b_reduce.reshape(1, H), u1, us2, ui2)

  return jnp.concatenate([fu[:NU_REAL], fi[:NI_REAL]], axis=0)


# speedup vs baseline: 1.0122x; 1.0122x over previous
import functools

import jax
import jax.numpy as jnp
from jax import lax
from jax.experimental import pallas as pl
from jax.experimental.pallas import tpu as pltpu
from jax.experimental.pallas import tpu_sc as plsc

NU_REAL = 50001
NI_REAL = 50000
H = 64
HH = 32
E = 800000

NC = 2
NS = 16

NPAD = 50176
STRIPE = NPAD // NS
ZROWS = 784
E_TILE = 50176
EPAD = E_TILE * NS
CHUNK = 64
NCHUNK = E_TILE // CHUNK


NBUF = 8
SBCH = NBUF * CHUNK
NSB = E_TILE // SBCH
ER = EPAD // CHUNK
TROWS = E_TILE // CHUNK


def _spmm_once(c, s, x, rr, cc, vv, yy, zz, acc, ibr, ibc, ibv, gath,
               sem_i, sem_g, sem_s):
  pltpu.sync_copy(zz, acc.at[pl.ds(s * STRIPE, STRIPE)])
  plsc.subcore_barrier()

  def issue_idx(sb, bb):
    r0 = s * TROWS + sb * NBUF
    pltpu.async_copy(rr.at[pl.ds(r0, NBUF)], ibr.at[bb], sem_i)
    pltpu.async_copy(vv.at[pl.ds(r0, NBUF)], ibv.at[bb], sem_i)
    pltpu.async_copy(cc.at[pl.ds(r0, NBUF)], ibc.at[bb], sem_i)

  def wait_idx(bb):
    pltpu.make_async_copy(rr.at[pl.ds(0, NBUF)], ibr.at[bb], sem_i).wait()
    pltpu.make_async_copy(vv.at[pl.ds(0, NBUF)], ibv.at[bb], sem_i).wait()
    pltpu.make_async_copy(cc.at[pl.ds(0, NBUF)], ibc.at[bb], sem_i).wait()

  def wait_scatter(j):
    pltpu.make_async_copy(gath.at[j], acc.at[ibr.at[0, j]],
                          sem_s.at[j]).wait()

  issue_idx(0, 0)

  def sb_body(sb, carry):
    bb = lax.rem(sb, 2)
    wait_idx(bb)

    @pl.when(sb < NSB - 1)
    def _():
      issue_idx(sb + 1, 1 - bb)

    for j in range(NBUF):
      @pl.when(sb > 0)
      def _():
        wait_scatter(j)

      @pl.when(c == 0)
      def _():
        pltpu.async_copy(x.at[0].at[ibc.at[bb, j]], gath.at[j], sem_g.at[j])

      @pl.when(c == 1)
      def _():
        pltpu.async_copy(x.at[1].at[ibc.at[bb, j]], gath.at[j], sem_g.at[j])

    for j in range(NBUF):
      pltpu.make_async_copy(x.at[0].at[ibc.at[bb, j]], gath.at[j],
                            sem_g.at[j]).wait()

      def scale16(t, carry2):
        vv16 = ibv[bb, j, pl.ds(t * 16, 16)]
        for u in range(16):
          e = t * 16 + u
          val = vv16[u]
          gath[j, e, pl.ds(0, 16)] = gath[j, e, pl.ds(0, 16)] * val
          gath[j, e, pl.ds(16, 16)] = gath[j, e, pl.ds(16, 16)] * val
        return carry2

      lax.fori_loop(0, CHUNK // 16, scale16, 0)
      pltpu.async_copy(gath.at[j], acc.at[ibr.at[bb, j]], sem_s.at[j],
                       add=True)
    return carry

  lax.fori_loop(0, NSB, sb_body, 0)
  for j in range(NBUF):
    wait_scatter(j)
  plsc.subcore_barrier()
  @pl.when(c == 0)
  def _():
    pltpu.sync_copy(acc.at[pl.ds(s * STRIPE, STRIPE)],
                    yy.at[pl.ds(s * STRIPE, STRIPE), pl.ds(0, HH)])

  @pl.when(c == 1)
  def _():
    pltpu.sync_copy(acc.at[pl.ds(s * STRIPE, STRIPE)],
                    yy.at[pl.ds(s * STRIPE, STRIPE), pl.ds(HH, HH)])

  plsc.subcore_barrier()


def _sc_pair_body(xu, xi, zz, r1, c1, v1, r2, c2, v2,
                  y1, y2, acc, ibr, ibc, ibv, gath,
                  sem_i, sem_g, sem_s):
  c = lax.axis_index("c")
  s = lax.axis_index("s")
  args = (acc, ibr, ibc, ibv, gath, sem_i, sem_g, sem_s)
  _spmm_once(c, s, xu, r1, c1, v1, y1, zz, *args)
  _spmm_once(c, s, xi, r2, c2, v2, y2, zz, *args)


def _sc_one_body(xu, zz, r3, c3, v3, y3, acc, ibr, ibc, ibv, gath,
                 sem_i, sem_g, sem_s):
  c = lax.axis_index("c")
  s = lax.axis_index("s")
  args = (acc, ibr, ibc, ibv, gath, sem_i, sem_g, sem_s)
  _spmm_once(c, s, xu, r3, c3, v3, y3, zz, *args)


_Y = jax.ShapeDtypeStruct((NPAD, H), jnp.float32)
_MESH = plsc.VectorSubcoreMesh(core_axis_name="c", subcore_axis_name="s",
                               num_cores=NC, num_subcores=NS)
_SCRATCH = [
    pltpu.VMEM_SHARED((NPAD, HH), jnp.float32),
    pltpu.VMEM((2, NBUF, CHUNK), jnp.int32),
    pltpu.VMEM((2, NBUF, CHUNK), jnp.int32),
    pltpu.VMEM((2, NBUF, CHUNK), jnp.float32),
    pltpu.VMEM((NBUF, CHUNK, HH), jnp.float32),
    pltpu.SemaphoreType.DMA,
    pltpu.SemaphoreType.DMA((NBUF,)),
    pltpu.SemaphoreType.DMA((NBUF,)),
]

_sc_pair = pl.kernel(
    _sc_pair_body,
    out_type=(_Y, _Y),
    mesh=_MESH,
    scratch_types=list(_SCRATCH),
    compiler_params=pltpu.CompilerParams(use_tc_tiling_on_sc=False),
)

_sc_one = pl.kernel(
    _sc_one_body,
    out_type=_Y,
    mesh=_MESH,
    scratch_types=list(_SCRATCH),
    compiler_params=pltpu.CompilerParams(use_tc_tiling_on_sc=False),
)


_BR = 1568
_R = NPAD // _BR


def _sums_body(x_ref, o_ref, acc_ref):
  r = pl.program_id(0)

  @pl.when(r == 0)
  def _():
    acc_ref[0] = 0.0
    acc_ref[1] = 0.0

  x = x_ref[...]
  acc_ref[0] += jnp.sum(x)
  acc_ref[1] += jnp.sum(x * x)

  @pl.when(r == _R - 1)
  def _():
    o_ref[0] = acc_ref[0]
    o_ref[1] = acc_ref[1]


def _apply_norm_body(ms_ref, x_ref, o_ref):
  mean = ms_ref[0, 0]
  scale = ms_ref[0, 1]
  y = (x_ref[...] - mean) * scale
  o_ref[0] = y[:, :HH]
  o_ref[1] = y[:, HH:]


def _normalize_split(x_pad, count):
  sums = pl.pallas_call(
      _sums_body,
      grid=(_R,),
      in_specs=[pl.BlockSpec((_BR, H), lambda r: (r, 0))],
      out_specs=pl.BlockSpec(memory_space=pltpu.SMEM),
      out_shape=jax.ShapeDtypeStruct((2,), jnp.float32),
      scratch_shapes=[pltpu.SMEM((2,), jnp.float32)],
  )(x_pad)
  mean = sums[0] / count
  var = sums[1] / count - mean * mean
  ms = jnp.stack([mean, 0.1 * lax.rsqrt(var)]).reshape(1, 2)
  return pl.pallas_call(
      _apply_norm_body,
      grid=(_R,),
      in_specs=[pl.BlockSpec(memory_space=pltpu.SMEM),
                pl.BlockSpec((_BR, H), lambda r: (r, 0))],
      out_specs=pl.BlockSpec((NC, _BR, HH), lambda r: (0, r, 0)),
      out_shape=jax.ShapeDtypeStruct((NC, NPAD, HH), jnp.float32),
  )(ms, x_pad)


def _attw(att, h0_ref, h1_ref, h2_ref):
  p0 = jnp.concatenate([h0_ref[0], h0_ref[1]], axis=1)
  p1 = h1_ref[...]
  p2 = h2_ref[...]
  cat = jnp.concatenate([p0, p1, p2], axis=1)
  logits = jnp.dot(cat, att, preferred_element_type=jnp.float32)
  m = jnp.max(logits, axis=1, keepdims=True)
  ex = jnp.exp(logits - m)
  w = ex / jnp.sum(ex, axis=1, keepdims=True)
  return w, p0, p1, p2


def _fuse_u_body(att_ref, u0_ref, us_ref, ui_ref, u1_ref):
  w, _, p1, p2 = _attw(att_ref[...], u0_ref, us_ref, ui_ref)
  for cc in range(NC):
    u1_ref[cc] = (w[:, 0:1] * u0_ref[cc]
                  + w[:, 1:2] * p1[:, cc * HH:(cc + 1) * HH]
                  + w[:, 2:3] * p2[:, cc * HH:(cc + 1) * HH])


def _fuse_u(att1, u0, us1, ui1):
  half = pl.BlockSpec((NC, _BR, HH), lambda r: (0, r, 0))
  full = pl.BlockSpec((_BR, H), lambda r: (r, 0))
  return pl.pallas_call(
      _fuse_u_body,
      grid=(_R,),
      in_specs=[pl.BlockSpec((3 * H, 3), lambda r: (0, 0)),
                half, full, full],
      out_specs=half,
      out_shape=jax.ShapeDtypeStruct((NC, NPAD, HH), jnp.float32),
  )(att1, u0, us1, ui1)


def _fuse_i_body(i0_ref, iu_ref, i1_ref):
  for cc in range(NC):
    i1_ref[cc] = 0.5 * (i0_ref[cc] + iu_ref[:, cc * HH:(cc + 1) * HH])


def _fuse_i(i0, iu1):
  half = pl.BlockSpec((NC, _BR, HH), lambda r: (0, r, 0))
  full = pl.BlockSpec((_BR, H), lambda r: (r, 0))
  return pl.pallas_call(
      _fuse_i_body,
      grid=(_R,),
      in_specs=[half, full],
      out_specs=half,
      out_shape=jax.ShapeDtypeStruct((NC, NPAD, HH), jnp.float32),
  )(i0, iu1)


def _final_u_body(att_ref, w_ref, b_ref, u1_ref, us_ref, ui_ref, fu_ref):
  wr = w_ref[...]
  b = b_ref[...]
  w, u1f, usf, uif = _attw(att_ref[...], u1_ref, us_ref, ui_ref)
  u2f = w[:, 0:1] * u1f + w[:, 1:2] * usf + w[:, 2:3] * uif
  zu = (jnp.dot(u1f, wr[:H], preferred_element_type=jnp.float32)
        + jnp.dot(u2f, wr[H:], preferred_element_type=jnp.float32) + b)
  fu_ref[...] = 1.0 / (1.0 + jnp.exp(-zu))


def _final_u(att2, w_reduce, b_reduce, u1, us2, ui2):
  half = pl.BlockSpec((NC, _BR, HH), lambda r: (0, r, 0))
  full = pl.BlockSpec((_BR, H), lambda r: (r, 0))
  return pl.pallas_call(
      _final_u_body,
      grid=(_R,),
      in_specs=[pl.BlockSpec((3 * H, 3), lambda r: (0, 0)),
                pl.BlockSpec((2 * H, H), lambda r: (0, 0)),
                pl.BlockSpec((1, H), lambda r: (0, 0)),
                half, full, full],
      out_specs=full,
      out_shape=jax.ShapeDtypeStruct((NPAD, H), jnp.float32),
  )(att2, w_reduce, b_reduce, u1, us2, ui2)


def _final_i_body(w_ref, b_ref, i1_ref, iu_ref, fi_ref):
  wr = w_ref[...]
  b = b_ref[...]
  i1f = jnp.concatenate([i1_ref[0], i1_ref[1]], axis=1)
  i2f = 0.5 * (i1f + iu_ref[...])
  zi = (jnp.dot(i1f, wr[:H], preferred_element_type=jnp.float32)
        + jnp.dot(i2f, wr[H:], preferred_element_type=jnp.float32) + b)
  fi_ref[...] = 1.0 / (1.0 + jnp.exp(-zi))


def _final_i(w_reduce, b_reduce, i1, iu2):
  half = pl.BlockSpec((NC, _BR, HH), lambda r: (0, r, 0))
  full = pl.BlockSpec((_BR, H), lambda r: (r, 0))
  return pl.pallas_call(
      _final_i_body,
      grid=(_R,),
      in_specs=[pl.BlockSpec((2 * H, H), lambda r: (0, 0)),
                pl.BlockSpec((1, H), lambda r: (0, 0)),
                half, full],
      out_specs=full,
      out_shape=jax.ShapeDtypeStruct((NPAD, H), jnp.float32),
  )(w_reduce, b_reduce, i1, iu2)


def _prep_edges(idx, vals):
  rows = idx[:, 0].astype(jnp.int32)
  cols = idx[:, 1].astype(jnp.int32)
  pad = EPAD - E
  rows_p = jnp.concatenate([rows, jnp.full((pad,), NPAD - 1, jnp.int32)])
  cols_p = jnp.concatenate([cols, jnp.zeros((pad,), jnp.int32)])
  vals_p = jnp.concatenate([vals, jnp.zeros((pad,), jnp.float32)])
  return (rows_p.reshape(ER, CHUNK), cols_p.reshape(ER, CHUNK),
          vals_p.reshape(ER, CHUNK))


def kernel(user_embedding, item_embedding, att1, att2, W_reduce, b_reduce,
           social_neighbors_indices, social_neighbors_values,
           consumed_items_indices, consumed_items_values,
           item_customer_indices, item_customer_values):
  ue = jnp.zeros((NPAD, H), jnp.float32).at[:NU_REAL].set(user_embedding)
  ie = jnp.zeros((NPAD, H), jnp.float32).at[:NI_REAL].set(item_embedding)

  u0 = _normalize_split(ue, float(NU_REAL * H))
  i0 = _normalize_split(ie, float(NI_REAL * H))

  r1, c1, v1 = _prep_edges(social_neighbors_indices, social_neighbors_values)
  r2, c2, v2 = _prep_edges(consumed_items_indices, consumed_items_values)
  r3, c3, v3 = _prep_edges(item_customer_indices, item_customer_values)

  zz = jnp.zeros((STRIPE, HH), jnp.float32)
  iu1 = _sc_one(u0, zz, r3, c3, v3)
  us1, ui1 = _sc_pair(u0, i0, zz, r1, c1, v1, r2, c2, v2)
  i1 = _fuse_i(i0, iu1)
  u1 = _fuse_u(att1, u0, us1, ui1)

  iu2 = _sc_one(u1, zz, r3, c3, v3)
  us2, ui2 = _sc_pair(u1, i1, zz, r1, c1, v1, r2, c2, v2)
  fi = _final_i(W_reduce, b_reduce.reshape(1, H), i1, iu2)
  fu = _final_u(att2, W_reduce, b_reduce.reshape(1, H), u1, us2, ui2)

  return jnp.concatenate([fu[:NU_REAL], fi[:NI_REAL]], axis=0)
